# baseline (speedup 1.0000x reference)
import functools

import jax
import jax.numpy as jnp
from jax import lax
from jax.experimental import pallas as pl
from jax.experimental.pallas import tpu as pltpu
from jax.experimental.pallas import tpu_sc as plsc

D = 128
NC = 2
NS = 16
NW = NC * NS
KCH = 80
WC = 8
TN = 1000
TE = 2000


def _dot(a, b):
    return jnp.dot(a, b, preferred_element_type=jnp.float32)


def _enc_body(x, A1, a1, A2, a2, A3, a3, Wa, Wb, h_o, hA_o, hB_o):
    h = jnp.maximum(_dot(x[...], A1[...]) + a1[...], 0.0)
    h = jnp.maximum(_dot(h, A2[...]) + a2[...], 0.0)
    h = _dot(h, A3[...]) + a3[...]
    h_o[...] = h
    hA_o[...] = _dot(h, Wa[...])
    hB_o[...] = _dot(h, Wb[...])


def _edge_enc_body(ea, E1, e1, E2, e2, E3, e3, e_o):
    h = jnp.maximum(_dot(ea[...], E1[...]) + e1[...], 0.0)
    h = jnp.maximum(_dot(h, E2[...]) + e2[...], 0.0)
    e_o[...] = (_dot(h, E3[...]) + e3[...]).astype(jnp.bfloat16)


def _edge_mlp_body(g, e, W1cP, b1P, W2P, b2, W3, b3, msg_o):
    xa = g[...]
    lo = lax.bitcast_convert_type(xa << 16, jnp.float32)
    hi = lax.bitcast_convert_type(xa & jnp.int32(-65536), jnp.float32)
    gf = jnp.concatenate([lo, hi], axis=-1)
    ec = jnp.dot(e[...], W1cP[...].astype(jnp.bfloat16),
                 preferred_element_type=jnp.float32)
    z = jnp.maximum(gf + ec + b1P[...], 0.0)
    z = jnp.maximum(_dot(z, W2P[...]) + b2[...], 0.0)
    msg_o[...] = _dot(z, W3[...]) + b3[...]


def _node_mid_body(h, S, cnt, V1a, V1b, c1, V2, c2, V3, c3, Wa, Wb,
                   h_o, hA_o, hB_o):
    s = S[0] + S[1]
    c = cnt[0, :, 0:1] + cnt[1, :, 0:1]
    aggr = s * (1.0 / jnp.maximum(c, 1.0))
    hh = h[...]
    u = jnp.maximum(_dot(hh, V1a[...]) + _dot(aggr, V1b[...]) + c1[...], 0.0)
    u = jnp.maximum(_dot(u, V2[...]) + c2[...], 0.0)
    u = _dot(u, V3[...]) + c3[...]
    hn = hh + u
    h_o[...] = hn
    hA_o[...] = _dot(hn, Wa[...])
    hB_o[...] = _dot(hn, Wb[...])


def _node_last_body(h, S, cnt, V1a, V1b, c1, V2, c2, V3, c3,
                    U1, d1, U2, d2, U3, d3, dec_o):
    s = S[0] + S[1]
    c = cnt[0, :, 0:1] + cnt[1, :, 0:1]
    aggr = s * (1.0 / jnp.maximum(c, 1.0))
    hh = h[...]
    u = jnp.maximum(_dot(hh, V1a[...]) + _dot(aggr, V1b[...]) + c1[...], 0.0)
    u = jnp.maximum(_dot(u, V2[...]) + c2[...], 0.0)
    hn = hh + _dot(u, V3[...]) + c3[...]
    t = jnp.maximum(_dot(hn, U1[...]) + d1[...], 0.0)
    t = jnp.maximum(_dot(t, U2[...]) + d2[...], 0.0)
    dec_o[...] = _dot(t, U3[...]) + d3[...]


def _pack_bf16(a):
    n, d = a.shape
    return lax.bitcast_convert_type(a.reshape(n, d // 2, 2), jnp.int32)


def _unpack_bf16(a):
    n, dw = a.shape
    return lax.bitcast_convert_type(a, jnp.bfloat16).reshape(n, 2 * dw)


def _sc_gather(hA, hB, src, dst):
    E = src.shape[0]
    per_w = E // NW
    KG = 80
    nch = per_w // KG
    npair = nch // 2
    mesh = plsc.VectorSubcoreMesh(core_axis_name="c", subcore_axis_name="s")

    @functools.partial(
        pl.kernel, mesh=mesh,
        out_type=jax.ShapeDtypeStruct((E, D // 2), jnp.int32),
        scratch_types=[
            pltpu.VMEM((2, KG), jnp.int32),
            pltpu.VMEM((2, KG), jnp.int32),
            pltpu.VMEM((2, KG, D), jnp.float32),
            pltpu.VMEM((2, KG, D), jnp.float32),
            pltpu.VMEM((2, KG, D // 2), jnp.int32),
            pltpu.SemaphoreType.DMA,
            pltpu.SemaphoreType.DMA,
            pltpu.SemaphoreType.DMA,
            pltpu.SemaphoreType.DMA,
            pltpu.SemaphoreType.DMA,
            pltpu.SemaphoreType.DMA,
        ],
    )
    def k(hA_h, hB_h, src_h, dst_h, g_h, ia2, ib2, ba2, bb2, gpk,
          si0, si1, sg0, sg1, sw0, sw1):
        wid = lax.axis_index("s") * NC + lax.axis_index("c")
        base = wid * per_w
        si = (si0, si1)
        sg = (sg0, sg1)
        sw = (sw0, sw1)

        def idx_drain(sem):
            pltpu.make_async_copy(dst_h.at[pl.ds(0, KG)], ia2.at[0], sem).wait()
            pltpu.make_async_copy(src_h.at[pl.ds(0, KG)], ib2.at[0], sem).wait()

        def gat_drain(sem):
            pltpu.make_async_copy(hA_h.at[pl.ds(0, KG)], ba2.at[0], sem).wait()
            pltpu.make_async_copy(hA_h.at[pl.ds(0, KG)], bb2.at[0], sem).wait()

        def out_drain(sem):
            pltpu.make_async_copy(gpk.at[0], g_h.at[pl.ds(0, KG)], sem).wait()

        pltpu.sync_copy(dst_h.at[pl.ds(base, KG)], ia2.at[0])
        pltpu.sync_copy(src_h.at[pl.ds(base, KG)], ib2.at[0])
        pltpu.async_copy(hA_h.at[ia2.at[0]], ba2.at[0], sg0)
        pltpu.async_copy(hB_h.at[ib2.at[0]], bb2.at[0], sg0)
        off1 = pl.multiple_of(base + KG, 8)
        pltpu.async_copy(dst_h.at[pl.ds(off1, KG)], ia2.at[1], si1)
        pltpu.async_copy(src_h.at[pl.ds(off1, KG)], ib2.at[1], si1)

        def add_and_flush(p, off, sem_w):
            def row(r, c2):
                for c2i in range(D // 32):
                    s0 = pl.ds(c2i * 32, 16)
                    s1 = pl.ds(c2i * 32 + 16, 16)
                    a0 = ba2[p, r, s0] + bb2[p, r, s0]
                    a1 = ba2[p, r, s1] + bb2[p, r, s1]
                    i0 = lax.bitcast_convert_type(a0, jnp.int32)
                    i1 = lax.bitcast_convert_type(a1, jnp.int32)
                    w = jnp.bitwise_or(
                        lax.shift_right_logical(i0 + 32768, 16),
                        jnp.bitwise_and(i1 + 32768, jnp.int32(-65536)))
                    gpk[p, r, pl.ds(c2i * 16, 16)] = w
                return c2

            lax.fori_loop(0, KG, row, 0)
            pltpu.async_copy(gpk.at[p], g_h.at[pl.ds(off, KG)], sem_w)

        def body(jj, carry):
            j = 2 * jj
            idx_drain(si1)

            @pl.when(jj >= 1)
            def _():
                out_drain(sw1)

            pltpu.async_copy(hA_h.at[ia2.at[1]], ba2.at[1], sg1)
            pltpu.async_copy(hB_h.at[ib2.at[1]], bb2.at[1], sg1)
            gat_drain(sg0)

            @pl.when(j + 2 < nch)
            def _():
                off2 = pl.multiple_of(base + (j + 2) * KG, 8)
                pltpu.async_copy(dst_h.at[pl.ds(off2, KG)], ia2.at[0], si0)
                pltpu.async_copy(src_h.at[pl.ds(off2, KG)], ib2.at[0], si0)

            add_and_flush(0, pl.multiple_of(base + j * KG, 16), sw0)

            @pl.when(j + 2 < nch)
            def _():
                idx_drain(si0)
                out_drain(sw0)
                pltpu.async_copy(hA_h.at[ia2.at[0]], ba2.at[0], sg0)
                pltpu.async_copy(hB_h.at[ib2.at[0]], bb2.at[0], sg0)

            gat_drain(sg1)

            @pl.when(j + 3 < nch)
            def _():
                off3 = pl.multiple_of(base + (j + 3) * KG, 8)
                pltpu.async_copy(dst_h.at[pl.ds(off3, KG)], ia2.at[1], si1)
                pltpu.async_copy(src_h.at[pl.ds(off3, KG)], ib2.at[1], si1)

            add_and_flush(1, pl.multiple_of(base + (j + 1) * KG, 16), sw1)
            return carry

        lax.fori_loop(0, npair, body, 0)
        gat_drain(sg0)
        add_and_flush(0, pl.multiple_of(base + (nch - 1) * KG, 16), sw0)
        out_drain(sw1)
        out_drain(sw0)

    return k(hA, hB, src, dst)


def _sc_scatter(msg, dst, zeros_nd):
    E = msg.shape[0]
    n_pad = zeros_nd.shape[0]
    per_w = E // NW
    KS = 40
    nch = per_w // KS
    npair = nch // 2
    rows = n_pad // NS
    mesh = plsc.VectorSubcoreMesh(core_axis_name="c", subcore_axis_name="s")

    @functools.partial(
        pl.kernel, mesh=mesh,
        out_type=jax.ShapeDtypeStruct((NC, n_pad, D), jnp.float32),
        scratch_types=[
            pltpu.VMEM((2, KS, D), jnp.float32),
            pltpu.VMEM((2, KS), jnp.int32),
            pltpu.VMEM_SHARED((n_pad, D), jnp.float32),
            pltpu.SemaphoreType.DMA,
            pltpu.SemaphoreType.DMA,
            pltpu.SemaphoreType.DMA,
            pltpu.SemaphoreType.DMA,
        ],
    )
    def k(msg_h, dst_h, zeros_h, out_h, buf2, idx2, accum, sl0, sl1, ss0, ss1):
        cid = lax.axis_index("c")
        sid = lax.axis_index("s")
        wid = sid * NC + cid
        base = wid * per_w
        pltpu.sync_copy(zeros_h.at[pl.ds(sid * rows, rows)],
                        accum.at[pl.ds(sid * rows, rows)])
        plsc.subcore_barrier()

        pltpu.async_copy(dst_h.at[pl.ds(base, KS)], idx2.at[0], sl0)
        pltpu.async_copy(msg_h.at[pl.ds(base, KS)], buf2.at[0], sl0)

        def body(j, carry):
            p = lax.rem(j, 2)
            pltpu.make_async_copy(dst_h.at[pl.ds(0, KS)], idx2.at[0], sl0).wait()
            pltpu.make_async_copy(msg_h.at[pl.ds(0, KS)], buf2.at[0], sl0).wait()
            pltpu.async_copy(buf2.at[p], accum.at[idx2.at[p]], ss0, add=True)

            @pl.when(j >= 1)
            def _():
                pltpu.make_async_copy(buf2.at[0], accum.at[pl.ds(0, KS)],
                                      ss0).wait()

            @pl.when(j + 1 < nch)
            def _():
                off2 = pl.multiple_of(base + (j + 1) * KS, 8)
                pltpu.async_copy(dst_h.at[pl.ds(off2, KS)], idx2.at[1 - p], sl0)
                pltpu.async_copy(msg_h.at[pl.ds(off2, KS)], buf2.at[1 - p], sl0)

            return carry

        lax.fori_loop(0, nch, body, 0)
        pltpu.make_async_copy(buf2.at[0], accum.at[pl.ds(0, KS)], ss0).wait()
        plsc.subcore_barrier()
        pltpu.sync_copy(accum.at[pl.ds(sid * rows, rows)],
                        out_h.at[cid, pl.ds(sid * rows, rows)])

    return k(msg, dst, zeros_nd)


def _sc_counts(dst, ones_kw, zeros_nw):
    E = dst.shape[0]
    n_pad = zeros_nw.shape[0]
    per_w = E // NW
    nch = per_w // KCH
    rows = n_pad // NS
    mesh = plsc.VectorSubcoreMesh(core_axis_name="c", subcore_axis_name="s")

    @functools.partial(
        pl.kernel, mesh=mesh,
        out_type=jax.ShapeDtypeStruct((NC, n_pad, WC), jnp.float32),
        scratch_types=[
            pltpu.VMEM((KCH, WC), jnp.float32),
            pltpu.VMEM((KCH,), jnp.int32),
            pltpu.VMEM_SHARED((n_pad, WC), jnp.float32),
        ],
    )
    def k(dst_h, ones_h, zeros_h, out_h, ones_v, idx, accum):
        cid = lax.axis_index("c")
        sid = lax.axis_index("s")
        wid = sid * NC + cid
        pltpu.sync_copy(ones_h, ones_v)
        pltpu.sync_copy(zeros_h.at[pl.ds(sid * rows, rows)],
                        accum.at[pl.ds(sid * rows, rows)])
        plsc.subcore_barrier()

        def body(j, carry):
            off = pl.multiple_of(wid * per_w + j * KCH, 8)
            pltpu.sync_copy(dst_h.at[pl.ds(off, KCH)], idx)
            pltpu.sync_copy(ones_v, accum.at[idx], add=True)
            return carry

        lax.fori_loop(0, nch, body, 0)
        plsc.subcore_barrier()
        pltpu.sync_copy(accum.at[pl.ds(sid * rows, rows)],
                        out_h.at[cid, pl.ds(sid * rows, rows)])

    return k(dst, ones_kw, zeros_nw)


def kernel(x, edge_index, edge_attr, params):
    n_nodes = x.shape[0]
    E = edge_attr.shape[0]
    src = edge_index[0]
    dst = edge_index[1]

    blocks = params['blocks']
    n_grid = n_nodes // TN
    e_grid = E // TE

    def r2(w):
        return w

    def b2(b):
        return b.reshape(1, -1)

    (A1, a1), (A2, a2), (A3, a3) = params['node_enc']
    W1_0 = blocks[0]['edge_mlp'][0][0]
    Wa0, Wb0 = W1_0[:D], W1_0[D:2 * D]
    row_spec = pl.BlockSpec((TN, D), lambda i: (i, 0))
    wspec = pl.BlockSpec((D, D), lambda i: (0, 0))
    bspec = pl.BlockSpec((1, D), lambda i: (0, 0))
    h, hA, hB = pl.pallas_call(
        _enc_body,
        grid=(n_grid,),
        in_specs=[row_spec] + [wspec, bspec] * 3 + [wspec, wspec],
        out_specs=(row_spec, row_spec, row_spec),
        out_shape=(jax.ShapeDtypeStruct((n_nodes, D), jnp.float32),) * 3,
    )(x, A1, b2(a1), A2, b2(a2), A3, b2(a3), Wa0, Wb0)

    (E1, e1), (E2, e2), (E3, e3) = params['edge_enc']
    k_in = E1.shape[0]
    k_pad = 8
    ea8 = jnp.pad(edge_attr, ((0, 0), (0, k_pad - k_in)))
    E1p = jnp.pad(E1, ((0, k_pad - k_in), (0, 0)))
    erow_spec = pl.BlockSpec((TE, D), lambda i: (i, 0))
    ein_spec = pl.BlockSpec((TE, k_pad), lambda i: (i, 0))
    ew_spec = pl.BlockSpec((k_pad, D), lambda i: (0, 0))
    e = pl.pallas_call(
        _edge_enc_body,
        grid=(e_grid,),
        in_specs=[ein_spec, ew_spec, bspec, wspec, bspec, wspec, bspec],
        out_specs=erow_spec,
        out_shape=jax.ShapeDtypeStruct((E, D), jnp.bfloat16),
    )(ea8, E1p, b2(e1), E2, b2(e2), E3, b2(e3))

    n_pad = ((n_nodes + 8 * NS - 1) // (8 * NS)) * (8 * NS)
    ones_kw = jnp.ones((KCH, WC), jnp.float32)
    zeros_nw = jnp.zeros((n_pad, WC), jnp.float32)
    zeros_nd = jnp.zeros((n_pad, D), jnp.float32)
    cnt = _sc_counts(dst, ones_kw, zeros_nw)

    spart_spec = pl.BlockSpec((NC, TN, D), lambda i: (0, i, 0))
    cnt_spec = pl.BlockSpec((NC, TN, WC), lambda i: (0, i, 0))
    perm = jnp.array(
        [32 * (l // 16) + l % 16 for l in range(D // 2)]
        + [32 * (l // 16) + 16 + l % 16 for l in range(D // 2)], jnp.int32)
    g_spec = pl.BlockSpec((TE, D // 2), lambda i: (i, 0))

    for bi, blk in enumerate(blocks):
        (W1, w_b1), (W2, w_b2), (W3, w_b3) = blk['edge_mlp']
        W1c = W1[2 * D:]
        g3 = _sc_gather(hA, hB, src, dst)
        msg = pl.pallas_call(
            _edge_mlp_body,
            grid=(e_grid,),
            in_specs=[g_spec, erow_spec,
                      wspec, bspec, wspec, bspec, wspec, bspec],
            out_specs=erow_spec,
            out_shape=jax.ShapeDtypeStruct((E, D), jnp.float32),
        )(g3, e, W1c[:, perm], b2(w_b1)[:, perm], W2[perm, :],
          b2(w_b2), W3, b2(w_b3))

        S = _sc_scatter(msg, dst, zeros_nd)

        (V1, c1), (V2, c2), (V3, c3) = blk['node_mlp']
        V1a, V1b = V1[:D], V1[D:]
        if bi + 1 < len(blocks):
            W1n = blocks[bi + 1]['edge_mlp'][0][0]
            Wan, Wbn = W1n[:D], W1n[D:2 * D]
            h, hA, hB = pl.pallas_call(
                _node_mid_body,
                grid=(n_grid,),
                in_specs=[row_spec, spart_spec, cnt_spec,
                          wspec, wspec, bspec, wspec, bspec, wspec, bspec,
                          wspec, wspec],
                out_specs=(row_spec, row_spec, row_spec),
                out_shape=(jax.ShapeDtypeStruct((n_nodes, D), jnp.float32),) * 3,
            )(h, S, cnt, V1a, V1b, b2(c1), V2, b2(c2), V3, b2(c3), Wan, Wbn)
        else:
            (U1, d1), (U2, d2), (U3, d3) = params['decoder']
            out_dim = U3.shape[1]
            U3p = jnp.pad(U3, ((0, 0), (0, D - out_dim)))
            d3p = jnp.pad(d3, ((0, D - out_dim),))
            dec = pl.pallas_call(
                _node_last_body,
                grid=(n_grid,),
                in_specs=[row_spec, spart_spec, cnt_spec,
                          wspec, wspec, bspec, wspec, bspec, wspec, bspec,
                          wspec, bspec, wspec, bspec, wspec, bspec],
                out_specs=row_spec,
                out_shape=jax.ShapeDtypeStruct((n_nodes, D), jnp.float32),
            )(h, S, cnt, V1a, V1b, b2(c1), V2, b2(c2), V3, b2(c3),
              U1, b2(d1), U2, b2(d2), U3p, b2(d3p))

    return dec[:, :out_dim]


# speedup vs baseline: 1.1095x; 1.1095x over previous
import functools

import jax
import jax.numpy as jnp
from jax import lax
from jax.experimental import pallas as pl
from jax.experimental.pallas import tpu as pltpu
from jax.experimental.pallas import tpu_sc as plsc

D = 128
NC = 2
NS = 16
NW = NC * NS
KCH = 80
WC = 8
TN = 1000
TE = 2000


def _dot(a, b):
    return jnp.dot(a, b, preferred_element_type=jnp.float32)


def _enc_body(x, A1, a1, A2, a2, A3, a3, Wa, Wb, h_o, hA_o, hB_o):
    h = jnp.maximum(_dot(x[...], A1[...]) + a1[...], 0.0)
    h = jnp.maximum(_dot(h, A2[...]) + a2[...], 0.0)
    h = _dot(h, A3[...]) + a3[...]
    h_o[...] = h
    hA_o[...] = _dot(h, Wa[...])
    hB_o[...] = _dot(h, Wb[...])


def _edge_enc_body(ea, E1, e1, E2, e2, E3, e3, e_o):
    h = jnp.maximum(_dot(ea[...], E1[...]) + e1[...], 0.0)
    h = jnp.maximum(_dot(h, E2[...]) + e2[...], 0.0)
    e_o[...] = (_dot(h, E3[...]) + e3[...]).astype(jnp.bfloat16)


def _edge_mlp_body(g, e, W1cP, b1P, W2P, b2, W3, b3, msg_o):
    xa = g[...]
    lo = lax.bitcast_convert_type(xa << 16, jnp.float32)
    hi = lax.bitcast_convert_type(xa & jnp.int32(-65536), jnp.float32)
    gf = jnp.concatenate([lo, hi], axis=-1)
    ec = jnp.dot(e[...], W1cP[...].astype(jnp.bfloat16),
                 preferred_element_type=jnp.float32)
    z = jnp.maximum(gf + ec + b1P[...], 0.0)
    z = jnp.maximum(_dot(z, W2P[...]) + b2[...], 0.0)
    msg_o[...] = _dot(z, W3[...]) + b3[...]


def _node_mid_body(h, S, cnt, V1a, V1b, c1, V2, c2, V3, c3, Wa, Wb,
                   h_o, hA_o, hB_o):
    s = S[0] + S[1]
    c = cnt[0, :, 0:1] + cnt[1, :, 0:1]
    aggr = s * (1.0 / jnp.maximum(c, 1.0))
    hh = h[...]
    u = jnp.maximum(_dot(hh, V1a[...]) + _dot(aggr, V1b[...]) + c1[...], 0.0)
    u = jnp.maximum(_dot(u, V2[...]) + c2[...], 0.0)
    u = _dot(u, V3[...]) + c3[...]
    hn = hh + u
    h_o[...] = hn
    hA_o[...] = _dot(hn, Wa[...])
    hB_o[...] = _dot(hn, Wb[...])


def _node_last_body(h, S, cnt, V1a, V1b, c1, V2, c2, V3, c3,
                    U1, d1, U2, d2, U3, d3, dec_o):
    s = S[0] + S[1]
    c = cnt[0, :, 0:1] + cnt[1, :, 0:1]
    aggr = s * (1.0 / jnp.maximum(c, 1.0))
    hh = h[...]
    u = jnp.maximum(_dot(hh, V1a[...]) + _dot(aggr, V1b[...]) + c1[...], 0.0)
    u = jnp.maximum(_dot(u, V2[...]) + c2[...], 0.0)
    hn = hh + _dot(u, V3[...]) + c3[...]
    t = jnp.maximum(_dot(hn, U1[...]) + d1[...], 0.0)
    t = jnp.maximum(_dot(t, U2[...]) + d2[...], 0.0)
    dec_o[...] = _dot(t, U3[...]) + d3[...]


def _pack_bf16(a):
    n, d = a.shape
    return lax.bitcast_convert_type(a.reshape(n, d // 2, 2), jnp.int32)


def _unpack_bf16(a):
    n, dw = a.shape
    return lax.bitcast_convert_type(a, jnp.bfloat16).reshape(n, 2 * dw)


def _sc_gather(hA, hB, src, dst):
    E = src.shape[0]
    per_w = E // NW
    KG = 80
    nch = per_w // KG
    npair = nch // 2
    mesh = plsc.VectorSubcoreMesh(core_axis_name="c", subcore_axis_name="s")

    @functools.partial(
        pl.kernel, mesh=mesh,
        out_type=jax.ShapeDtypeStruct((E, D // 2), jnp.int32),
        scratch_types=[
            pltpu.VMEM((2, KG), jnp.int32),
            pltpu.VMEM((2, KG), jnp.int32),
            pltpu.VMEM((2, KG, D), jnp.float32),
            pltpu.VMEM((2, KG, D), jnp.float32),
            pltpu.VMEM((2, KG, D // 2), jnp.int32),
            pltpu.SemaphoreType.DMA,
            pltpu.SemaphoreType.DMA,
            pltpu.SemaphoreType.DMA,
            pltpu.SemaphoreType.DMA,
            pltpu.SemaphoreType.DMA,
            pltpu.SemaphoreType.DMA,
        ],
    )
    def k(hA_h, hB_h, src_h, dst_h, g_h, ia2, ib2, ba2, bb2, gpk,
          si0, si1, sg0, sg1, sw0, sw1):
        wid = lax.axis_index("s") * NC + lax.axis_index("c")
        base = wid * per_w
        si = (si0, si1)
        sg = (sg0, sg1)
        sw = (sw0, sw1)

        def idx_drain(sem):
            pltpu.make_async_copy(dst_h.at[pl.ds(0, KG)], ia2.at[0], sem).wait()
            pltpu.make_async_copy(src_h.at[pl.ds(0, KG)], ib2.at[0], sem).wait()

        def gat_drain(sem):
            pltpu.make_async_copy(hA_h.at[pl.ds(0, KG)], ba2.at[0], sem).wait()
            pltpu.make_async_copy(hA_h.at[pl.ds(0, KG)], bb2.at[0], sem).wait()

        def out_drain(sem):
            pltpu.make_async_copy(gpk.at[0], g_h.at[pl.ds(0, KG)], sem).wait()

        pltpu.sync_copy(dst_h.at[pl.ds(base, KG)], ia2.at[0])
        pltpu.sync_copy(src_h.at[pl.ds(base, KG)], ib2.at[0])
        pltpu.async_copy(hA_h.at[ia2.at[0]], ba2.at[0], sg0)
        pltpu.async_copy(hB_h.at[ib2.at[0]], bb2.at[0], sg0)
        off1 = pl.multiple_of(base + KG, 8)
        pltpu.async_copy(dst_h.at[pl.ds(off1, KG)], ia2.at[1], si1)
        pltpu.async_copy(src_h.at[pl.ds(off1, KG)], ib2.at[1], si1)

        def add_and_flush(p, off, sem_w):
            def row(r, c2):
                for c2i in range(D // 32):
                    s0 = pl.ds(c2i * 32, 16)
                    s1 = pl.ds(c2i * 32 + 16, 16)
                    a0 = ba2[p, r, s0] + bb2[p, r, s0]
                    a1 = ba2[p, r, s1] + bb2[p, r, s1]
                    i0 = lax.bitcast_convert_type(a0, jnp.int32)
                    i1 = lax.bitcast_convert_type(a1, jnp.int32)
                    w = jnp.bitwise_or(
                        lax.shift_right_logical(i0 + 32768, 16),
                        jnp.bitwise_and(i1 + 32768, jnp.int32(-65536)))
                    gpk[p, r, pl.ds(c2i * 16, 16)] = w
                return c2

            lax.fori_loop(0, KG, row, 0)
            pltpu.async_copy(gpk.at[p], g_h.at[pl.ds(off, KG)], sem_w)

        def body(jj, carry):
            j = 2 * jj
            idx_drain(si1)

            @pl.when(jj >= 1)
            def _():
                out_drain(sw1)

            pltpu.async_copy(hA_h.at[ia2.at[1]], ba2.at[1], sg1)
            pltpu.async_copy(hB_h.at[ib2.at[1]], bb2.at[1], sg1)
            gat_drain(sg0)

            @pl.when(j + 2 < nch)
            def _():
                off2 = pl.multiple_of(base + (j + 2) * KG, 8)
                pltpu.async_copy(dst_h.at[pl.ds(off2, KG)], ia2.at[0], si0)
                pltpu.async_copy(src_h.at[pl.ds(off2, KG)], ib2.at[0], si0)

            add_and_flush(0, pl.multiple_of(base + j * KG, 16), sw0)

            @pl.when(j + 2 < nch)
            def _():
                idx_drain(si0)
                out_drain(sw0)
                pltpu.async_copy(hA_h.at[ia2.at[0]], ba2.at[0], sg0)
                pltpu.async_copy(hB_h.at[ib2.at[0]], bb2.at[0], sg0)

            gat_drain(sg1)

            @pl.when(j + 3 < nch)
            def _():
                off3 = pl.multiple_of(base + (j + 3) * KG, 8)
                pltpu.async_copy(dst_h.at[pl.ds(off3, KG)], ia2.at[1], si1)
                pltpu.async_copy(src_h.at[pl.ds(off3, KG)], ib2.at[1], si1)

            add_and_flush(1, pl.multiple_of(base + (j + 1) * KG, 16), sw1)
            return carry

        lax.fori_loop(0, npair, body, 0)
        gat_drain(sg0)
        add_and_flush(0, pl.multiple_of(base + (nch - 1) * KG, 16), sw0)
        out_drain(sw1)
        out_drain(sw0)

    return k(hA, hB, src, dst)


def _sc_scatter(msg, dst, zeros_nd):
    E = msg.shape[0]
    n_pad = zeros_nd.shape[0]
    per_w = E // NW
    KS = 80
    nch = per_w // KS
    rows = n_pad // NS
    mesh = plsc.VectorSubcoreMesh(core_axis_name="c", subcore_axis_name="s")

    @functools.partial(
        pl.kernel, mesh=mesh,
        out_type=jax.ShapeDtypeStruct((NC, n_pad, D), jnp.float32),
        scratch_types=[
            pltpu.VMEM((2, KS, D), jnp.float32),
            pltpu.VMEM((2, KS), jnp.int32),
            pltpu.VMEM_SHARED((n_pad, D), jnp.float32),
            pltpu.SemaphoreType.DMA,
            pltpu.SemaphoreType.DMA,
            pltpu.SemaphoreType.DMA,
            pltpu.SemaphoreType.DMA,
        ],
    )
    def k(msg_h, dst_h, zeros_h, out_h, buf2, idx2, accum, sl0, sl1, ss0, ss1):
        cid = lax.axis_index("c")
        sid = lax.axis_index("s")
        wid = sid * NC + cid
        base = wid * per_w
        pltpu.sync_copy(zeros_h.at[pl.ds(sid * rows, rows)],
                        accum.at[pl.ds(sid * rows, rows)])
        plsc.subcore_barrier()

        pltpu.async_copy(dst_h.at[pl.ds(base, KS)], idx2.at[0], sl0)
        pltpu.async_copy(msg_h.at[pl.ds(base, KS)], buf2.at[0], sl0)

        def body(j, carry):
            p = lax.rem(j, 2)
            pltpu.make_async_copy(dst_h.at[pl.ds(0, KS)], idx2.at[0], sl0).wait()
            pltpu.make_async_copy(msg_h.at[pl.ds(0, KS)], buf2.at[0], sl0).wait()
            pltpu.async_copy(buf2.at[p], accum.at[idx2.at[p]], ss0, add=True)

            @pl.when(j >= 1)
            def _():
                pltpu.make_async_copy(buf2.at[0], accum.at[pl.ds(0, KS)],
                                      ss0).wait()

            @pl.when(j + 1 < nch)
            def _():
                off2 = pl.multiple_of(base + (j + 1) * KS, 8)
                pltpu.async_copy(dst_h.at[pl.ds(off2, KS)], idx2.at[1 - p], sl0)
                pltpu.async_copy(msg_h.at[pl.ds(off2, KS)], buf2.at[1 - p], sl0)

            return carry

        lax.fori_loop(0, nch, body, 0)
        pltpu.make_async_copy(buf2.at[0], accum.at[pl.ds(0, KS)], ss0).wait()
        plsc.subcore_barrier()
        pltpu.sync_copy(accum.at[pl.ds(sid * rows, rows)],
                        out_h.at[cid, pl.ds(sid * rows, rows)])

    return k(msg, dst, zeros_nd)


def _sc_counts(dst, ones_kw, zeros_nw):
    E = dst.shape[0]
    n_pad = zeros_nw.shape[0]
    per_w = E // NW
    nch = per_w // KCH
    rows = n_pad // NS
    mesh = plsc.VectorSubcoreMesh(core_axis_name="c", subcore_axis_name="s")

    @functools.partial(
        pl.kernel, mesh=mesh,
        out_type=jax.ShapeDtypeStruct((NC, n_pad, WC), jnp.float32),
        scratch_types=[
            pltpu.VMEM((KCH, WC), jnp.float32),
            pltpu.VMEM((KCH,), jnp.int32),
            pltpu.VMEM_SHARED((n_pad, WC), jnp.float32),
        ],
    )
    def k(dst_h, ones_h, zeros_h, out_h, ones_v, idx, accum):
        cid = lax.axis_index("c")
        sid = lax.axis_index("s")
        wid = sid * NC + cid
        pltpu.sync_copy(ones_h, ones_v)
        pltpu.sync_copy(zeros_h.at[pl.ds(sid * rows, rows)],
                        accum.at[pl.ds(sid * rows, rows)])
        plsc.subcore_barrier()

        def body(j, carry):
            off = pl.multiple_of(wid * per_w + j * KCH, 8)
            pltpu.sync_copy(dst_h.at[pl.ds(off, KCH)], idx)
            pltpu.sync_copy(ones_v, accum.at[idx], add=True)
            return carry

        lax.fori_loop(0, nch, body, 0)
        plsc.subcore_barrier()
        pltpu.sync_copy(accum.at[pl.ds(sid * rows, rows)],
                        out_h.at[cid, pl.ds(sid * rows, rows)])

    return k(dst, ones_kw, zeros_nw)


def kernel(x, edge_index, edge_attr, params):
    n_nodes = x.shape[0]
    E = edge_attr.shape[0]
    src = edge_index[0]
    dst = edge_index[1]

    blocks = params['blocks']
    n_grid = n_nodes // TN
    e_grid = E // TE

    def r2(w):
        return w

    def b2(b):
        return b.reshape(1, -1)

    (A1, a1), (A2, a2), (A3, a3) = params['node_enc']
    W1_0 = blocks[0]['edge_mlp'][0][0]
    Wa0, Wb0 = W1_0[:D], W1_0[D:2 * D]
    row_spec = pl.BlockSpec((TN, D), lambda i: (i, 0))
    wspec = pl.BlockSpec((D, D), lambda i: (0, 0))
    bspec = pl.BlockSpec((1, D), lambda i: (0, 0))
    h, hA, hB = pl.pallas_call(
        _enc_body,
        grid=(n_grid,),
        in_specs=[row_spec] + [wspec, bspec] * 3 + [wspec, wspec],
        out_specs=(row_spec, row_spec, row_spec),
        out_shape=(jax.ShapeDtypeStruct((n_nodes, D), jnp.float32),) * 3,
    )(x, A1, b2(a1), A2, b2(a2), A3, b2(a3), Wa0, Wb0)

    (E1, e1), (E2, e2), (E3, e3) = params['edge_enc']
    k_in = E1.shape[0]
    k_pad = 8
    ea8 = jnp.pad(edge_attr, ((0, 0), (0, k_pad - k_in)))
    E1p = jnp.pad(E1, ((0, k_pad - k_in), (0, 0)))
    erow_spec = pl.BlockSpec((TE, D), lambda i: (i, 0))
    ein_spec = pl.BlockSpec((TE, k_pad), lambda i: (i, 0))
    ew_spec = pl.BlockSpec((k_pad, D), lambda i: (0, 0))
    e = pl.pallas_call(
        _edge_enc_body,
        grid=(e_grid,),
        in_specs=[ein_spec, ew_spec, bspec, wspec, bspec, wspec, bspec],
        out_specs=erow_spec,
        out_shape=jax.ShapeDtypeStruct((E, D), jnp.bfloat16),
    )(ea8, E1p, b2(e1), E2, b2(e2), E3, b2(e3))

    n_pad = ((n_nodes + 8 * NS - 1) // (8 * NS)) * (8 * NS)
    ones_kw = jnp.ones((KCH, WC), jnp.float32)
    zeros_nw = jnp.zeros((n_pad, WC), jnp.float32)
    zeros_nd = jnp.zeros((n_pad, D), jnp.float32)
    cnt = _sc_counts(dst, ones_kw, zeros_nw)

    spart_spec = pl.BlockSpec((NC, TN, D), lambda i: (0, i, 0))
    cnt_spec = pl.BlockSpec((NC, TN, WC), lambda i: (0, i, 0))
    perm = jnp.array(
        [32 * (l // 16) + l % 16 for l in range(D // 2)]
        + [32 * (l // 16) + 16 + l % 16 for l in range(D // 2)], jnp.int32)
    g_spec = pl.BlockSpec((TE, D // 2), lambda i: (i, 0))

    for bi, blk in enumerate(blocks):
        (W1, w_b1), (W2, w_b2), (W3, w_b3) = blk['edge_mlp']
        W1c = W1[2 * D:]
        g3 = _sc_gather(hA, hB, src, dst)
        msg = pl.pallas_call(
            _edge_mlp_body,
            grid=(e_grid,),
            in_specs=[g_spec, erow_spec,
                      wspec, bspec, wspec, bspec, wspec, bspec],
            out_specs=erow_spec,
            out_shape=jax.ShapeDtypeStruct((E, D), jnp.float32),
        )(g3, e, W1c[:, perm], b2(w_b1)[:, perm], W2[perm, :],
          b2(w_b2), W3, b2(w_b3))

        S = _sc_scatter(msg, dst, zeros_nd)

        (V1, c1), (V2, c2), (V3, c3) = blk['node_mlp']
        V1a, V1b = V1[:D], V1[D:]
        if bi + 1 < len(blocks):
            W1n = blocks[bi + 1]['edge_mlp'][0][0]
            Wan, Wbn = W1n[:D], W1n[D:2 * D]
            h, hA, hB = pl.pallas_call(
                _node_mid_body,
                grid=(n_grid,),
                in_specs=[row_spec, spart_spec, cnt_spec,
                          wspec, wspec, bspec, wspec, bspec, wspec, bspec,
                          wspec, wspec],
                out_specs=(row_spec, row_spec, row_spec),
                out_shape=(jax.ShapeDtypeStruct((n_nodes, D), jnp.float32),) * 3,
            )(h, S, cnt, V1a, V1b, b2(c1), V2, b2(c2), V3, b2(c3), Wan, Wbn)
        else:
            (U1, d1), (U2, d2), (U3, d3) = params['decoder']
            out_dim = U3.shape[1]
            U3p = jnp.pad(U3, ((0, 0), (0, D - out_dim)))
            d3p = jnp.pad(d3, ((0, D - out_dim),))
            dec = pl.pallas_call(
                _node_last_body,
                grid=(n_grid,),
                in_specs=[row_spec, spart_spec, cnt_spec,
                          wspec, wspec, bspec, wspec, bspec, wspec, bspec,
                          wspec, bspec, wspec, bspec, wspec, bspec],
                out_specs=row_spec,
                out_shape=jax.ShapeDtypeStruct((n_nodes, D), jnp.float32),
            )(h, S, cnt, V1a, V1b, b2(c1), V2, b2(c2), V3, b2(c3),
              U1, b2(d1), U2, b2(d2), U3p, b2(d3p))

    return dec[:, :out_dim]


# speedup vs baseline: 1.2862x; 1.1592x over previous
import functools

import jax
import jax.numpy as jnp
from jax import lax
from jax.experimental import pallas as pl
from jax.experimental.pallas import tpu as pltpu
from jax.experimental.pallas import tpu_sc as plsc

D = 128
NC = 2
NS = 16
NW = NC * NS
KCH = 80
WC = 8
TN = 2000
TE = 4000


def _dot(a, b):
    return jnp.dot(a, b, preferred_element_type=jnp.float32)


def _enc_body(x, A1, a1, A2, a2, A3, a3, Wa, Wb, h_o, hA_o, hB_o):
    h = jnp.maximum(_dot(x[...], A1[...]) + a1[...], 0.0)
    h = jnp.maximum(_dot(h, A2[...]) + a2[...], 0.0)
    h = _dot(h, A3[...]) + a3[...]
    h_o[...] = h
    hA_o[...] = _dot(h, Wa[...])
    hB_o[...] = _dot(h, Wb[...])


def _edge_enc_body(ea, E1, e1, E2, e2, E3, e3, e_o):
    h = jnp.maximum(_dot(ea[...], E1[...]) + e1[...], 0.0)
    h = jnp.maximum(_dot(h, E2[...]) + e2[...], 0.0)
    e_o[...] = (_dot(h, E3[...]) + e3[...]).astype(jnp.bfloat16)


def _edge_mlp_body(g, e, W1cP, b1P, W2P, b2, W3, b3, msg_o):
    xa = g[...]
    lo = lax.bitcast_convert_type(xa << 16, jnp.float32)
    hi = lax.bitcast_convert_type(xa & jnp.int32(-65536), jnp.float32)
    gf = jnp.concatenate([lo, hi], axis=-1)
    ec = jnp.dot(e[...], W1cP[...].astype(jnp.bfloat16),
                 preferred_element_type=jnp.float32)
    z = jnp.maximum(gf + ec + b1P[...], 0.0)
    z = jnp.maximum(_dot(z, W2P[...]) + b2[...], 0.0)
    msg_o[...] = _dot(z, W3[...]) + b3[...]


def _node_mid_body(h, S, cnt, V1a, V1b, c1, V2, c2, V3, c3, Wa, Wb,
                   h_o, hA_o, hB_o):
    s = S[0] + S[1]
    c = cnt[0, :, 0:1] + cnt[1, :, 0:1]
    aggr = s * (1.0 / jnp.maximum(c, 1.0))
    hh = h[...]
    u = jnp.maximum(_dot(hh, V1a[...]) + _dot(aggr, V1b[...]) + c1[...], 0.0)
    u = jnp.maximum(_dot(u, V2[...]) + c2[...], 0.0)
    u = _dot(u, V3[...]) + c3[...]
    hn = hh + u
    h_o[...] = hn
    hA_o[...] = _dot(hn, Wa[...])
    hB_o[...] = _dot(hn, Wb[...])


def _node_last_body(h, S, cnt, V1a, V1b, c1, V2, c2, V3, c3,
                    U1, d1, U2, d2, U3, d3, dec_o):
    s = S[0] + S[1]
    c = cnt[0, :, 0:1] + cnt[1, :, 0:1]
    aggr = s * (1.0 / jnp.maximum(c, 1.0))
    hh = h[...]
    u = jnp.maximum(_dot(hh, V1a[...]) + _dot(aggr, V1b[...]) + c1[...], 0.0)
    u = jnp.maximum(_dot(u, V2[...]) + c2[...], 0.0)
    hn = hh + _dot(u, V3[...]) + c3[...]
    t = jnp.maximum(_dot(hn, U1[...]) + d1[...], 0.0)
    t = jnp.maximum(_dot(t, U2[...]) + d2[...], 0.0)
    dec_o[...] = _dot(t, U3[...]) + d3[...]


def _pack_bf16(a):
    n, d = a.shape
    return lax.bitcast_convert_type(a.reshape(n, d // 2, 2), jnp.int32)


def _unpack_bf16(a):
    n, dw = a.shape
    return lax.bitcast_convert_type(a, jnp.bfloat16).reshape(n, 2 * dw)


def _sc_gather(hA, hB, src, dst):
    E = src.shape[0]
    per_w = E // NW
    KG = 80
    nch = per_w // KG
    npair = nch // 2
    mesh = plsc.VectorSubcoreMesh(core_axis_name="c", subcore_axis_name="s")

    @functools.partial(
        pl.kernel, mesh=mesh,
        out_type=jax.ShapeDtypeStruct((E, D // 2), jnp.int32),
        scratch_types=[
            pltpu.VMEM((2, KG), jnp.int32),
            pltpu.VMEM((2, KG), jnp.int32),
            pltpu.VMEM((2, KG, D), jnp.float32),
            pltpu.VMEM((2, KG, D), jnp.float32),
            pltpu.VMEM((2, KG, D // 2), jnp.int32),
            pltpu.SemaphoreType.DMA,
            pltpu.SemaphoreType.DMA,
            pltpu.SemaphoreType.DMA,
            pltpu.SemaphoreType.DMA,
            pltpu.SemaphoreType.DMA,
            pltpu.SemaphoreType.DMA,
        ],
    )
    def k(hA_h, hB_h, src_h, dst_h, g_h, ia2, ib2, ba2, bb2, gpk,
          si0, si1, sg0, sg1, sw0, sw1):
        wid = lax.axis_index("s") * NC + lax.axis_index("c")
        base = wid * per_w
        si = (si0, si1)
        sg = (sg0, sg1)
        sw = (sw0, sw1)

        def idx_drain(sem):
            pltpu.make_async_copy(dst_h.at[pl.ds(0, KG)], ia2.at[0], sem).wait()
            pltpu.make_async_copy(src_h.at[pl.ds(0, KG)], ib2.at[0], sem).wait()

        def gat_drain(sem):
            pltpu.make_async_copy(hA_h.at[pl.ds(0, KG)], ba2.at[0], sem).wait()
            pltpu.make_async_copy(hA_h.at[pl.ds(0, KG)], bb2.at[0], sem).wait()

        def out_drain(sem):
            pltpu.make_async_copy(gpk.at[0], g_h.at[pl.ds(0, KG)], sem).wait()

        pltpu.sync_copy(dst_h.at[pl.ds(base, KG)], ia2.at[0])
        pltpu.sync_copy(src_h.at[pl.ds(base, KG)], ib2.at[0])
        pltpu.async_copy(hA_h.at[ia2.at[0]], ba2.at[0], sg0)
        pltpu.async_copy(hB_h.at[ib2.at[0]], bb2.at[0], sg0)
        off1 = pl.multiple_of(base + KG, 8)
        pltpu.async_copy(dst_h.at[pl.ds(off1, KG)], ia2.at[1], si1)
        pltpu.async_copy(src_h.at[pl.ds(off1, KG)], ib2.at[1], si1)

        def add_and_flush(p, off, sem_w):
            def row(r, c2):
                for c2i in range(D // 32):
                    s0 = pl.ds(c2i * 32, 16)
                    s1 = pl.ds(c2i * 32 + 16, 16)
                    a0 = ba2[p, r, s0] + bb2[p, r, s0]
                    a1 = ba2[p, r, s1] + bb2[p, r, s1]
                    i0 = lax.bitcast_convert_type(a0, jnp.int32)
                    i1 = lax.bitcast_convert_type(a1, jnp.int32)
                    w = jnp.bitwise_or(
                        lax.shift_right_logical(i0 + 32768, 16),
                        jnp.bitwise_and(i1 + 32768, jnp.int32(-65536)))
                    gpk[p, r, pl.ds(c2i * 16, 16)] = w
                return c2

            lax.fori_loop(0, KG, row, 0)
            pltpu.async_copy(gpk.at[p], g_h.at[pl.ds(off, KG)], sem_w)

        def body(jj, carry):
            j = 2 * jj
            idx_drain(si1)

            @pl.when(jj >= 1)
            def _():
                out_drain(sw1)

            pltpu.async_copy(hA_h.at[ia2.at[1]], ba2.at[1], sg1)
            pltpu.async_copy(hB_h.at[ib2.at[1]], bb2.at[1], sg1)
            gat_drain(sg0)

            @pl.when(j + 2 < nch)
            def _():
                off2 = pl.multiple_of(base + (j + 2) * KG, 8)
                pltpu.async_copy(dst_h.at[pl.ds(off2, KG)], ia2.at[0], si0)
                pltpu.async_copy(src_h.at[pl.ds(off2, KG)], ib2.at[0], si0)

            add_and_flush(0, pl.multiple_of(base + j * KG, 16), sw0)

            @pl.when(j + 2 < nch)
            def _():
                idx_drain(si0)
                out_drain(sw0)
                pltpu.async_copy(hA_h.at[ia2.at[0]], ba2.at[0], sg0)
                pltpu.async_copy(hB_h.at[ib2.at[0]], bb2.at[0], sg0)

            gat_drain(sg1)

            @pl.when(j + 3 < nch)
            def _():
                off3 = pl.multiple_of(base + (j + 3) * KG, 8)
                pltpu.async_copy(dst_h.at[pl.ds(off3, KG)], ia2.at[1], si1)
                pltpu.async_copy(src_h.at[pl.ds(off3, KG)], ib2.at[1], si1)

            add_and_flush(1, pl.multiple_of(base + (j + 1) * KG, 16), sw1)
            return carry

        lax.fori_loop(0, npair, body, 0)
        gat_drain(sg0)
        add_and_flush(0, pl.multiple_of(base + (nch - 1) * KG, 16), sw0)
        out_drain(sw1)
        out_drain(sw0)

    return k(hA, hB, src, dst)


def _sc_scatter(msg, dst, zeros_nd):
    E = msg.shape[0]
    n_pad = zeros_nd.shape[0]
    per_w = E // NW
    KS = 80
    nch = per_w // KS
    rows = n_pad // NS
    mesh = plsc.VectorSubcoreMesh(core_axis_name="c", subcore_axis_name="s")

    @functools.partial(
        pl.kernel, mesh=mesh,
        out_type=jax.ShapeDtypeStruct((NC, n_pad, D), jnp.float32),
        scratch_types=[
            pltpu.VMEM((2, KS, D), jnp.float32),
            pltpu.VMEM((2, KS), jnp.int32),
            pltpu.VMEM_SHARED((n_pad, D), jnp.float32),
            pltpu.SemaphoreType.DMA,
            pltpu.SemaphoreType.DMA,
            pltpu.SemaphoreType.DMA,
            pltpu.SemaphoreType.DMA,
        ],
    )
    def k(msg_h, dst_h, zeros_h, out_h, buf2, idx2, accum, sl0, sl1, ss0, ss1):
        cid = lax.axis_index("c")
        sid = lax.axis_index("s")
        wid = sid * NC + cid
        base = wid * per_w
        pltpu.sync_copy(zeros_h.at[pl.ds(sid * rows, rows)],
                        accum.at[pl.ds(sid * rows, rows)])
        plsc.subcore_barrier()

        pltpu.async_copy(dst_h.at[pl.ds(base, KS)], idx2.at[0], sl0)
        pltpu.async_copy(msg_h.at[pl.ds(base, KS)], buf2.at[0], sl0)

        def body(j, carry):
            p = lax.rem(j, 2)
            pltpu.make_async_copy(dst_h.at[pl.ds(0, KS)], idx2.at[0], sl0).wait()
            pltpu.make_async_copy(msg_h.at[pl.ds(0, KS)], buf2.at[0], sl0).wait()
            pltpu.async_copy(buf2.at[p], accum.at[idx2.at[p]], ss0, add=True)

            @pl.when(j >= 1)
            def _():
                pltpu.make_async_copy(buf2.at[0], accum.at[pl.ds(0, KS)],
                                      ss0).wait()

            @pl.when(j + 1 < nch)
            def _():
                off2 = pl.multiple_of(base + (j + 1) * KS, 8)
                pltpu.async_copy(dst_h.at[pl.ds(off2, KS)], idx2.at[1 - p], sl0)
                pltpu.async_copy(msg_h.at[pl.ds(off2, KS)], buf2.at[1 - p], sl0)

            return carry

        lax.fori_loop(0, nch, body, 0)
        pltpu.make_async_copy(buf2.at[0], accum.at[pl.ds(0, KS)], ss0).wait()
        plsc.subcore_barrier()
        pltpu.sync_copy(accum.at[pl.ds(sid * rows, rows)],
                        out_h.at[cid, pl.ds(sid * rows, rows)])

    return k(msg, dst, zeros_nd)


def _sc_counts(dst, ones_kw, zeros_nw):
    E = dst.shape[0]
    n_pad = zeros_nw.shape[0]
    per_w = E // NW
    nch = per_w // KCH
    rows = n_pad // NS
    mesh = plsc.VectorSubcoreMesh(core_axis_name="c", subcore_axis_name="s")

    @functools.partial(
        pl.kernel, mesh=mesh,
        out_type=jax.ShapeDtypeStruct((NC, n_pad, WC), jnp.float32),
        scratch_types=[
            pltpu.VMEM((KCH, WC), jnp.float32),
            pltpu.VMEM((KCH,), jnp.int32),
            pltpu.VMEM_SHARED((n_pad, WC), jnp.float32),
        ],
    )
    def k(dst_h, ones_h, zeros_h, out_h, ones_v, idx, accum):
        cid = lax.axis_index("c")
        sid = lax.axis_index("s")
        wid = sid * NC + cid
        pltpu.sync_copy(ones_h, ones_v)
        pltpu.sync_copy(zeros_h.at[pl.ds(sid * rows, rows)],
                        accum.at[pl.ds(sid * rows, rows)])
        plsc.subcore_barrier()

        def body(j, carry):
            off = pl.multiple_of(wid * per_w + j * KCH, 8)
            pltpu.sync_copy(dst_h.at[pl.ds(off, KCH)], idx)
            pltpu.sync_copy(ones_v, accum.at[idx], add=True)
            return carry

        lax.fori_loop(0, nch, body, 0)
        plsc.subcore_barrier()
        pltpu.sync_copy(accum.at[pl.ds(sid * rows, rows)],
                        out_h.at[cid, pl.ds(sid * rows, rows)])

    return k(dst, ones_kw, zeros_nw)


def kernel(x, edge_index, edge_attr, params):
    n_nodes = x.shape[0]
    E = edge_attr.shape[0]
    src = edge_index[0]
    dst = edge_index[1]

    blocks = params['blocks']
    n_grid = n_nodes // TN
    e_grid = E // TE

    def r2(w):
        return w

    def b2(b):
        return b.reshape(1, -1)

    (A1, a1), (A2, a2), (A3, a3) = params['node_enc']
    W1_0 = blocks[0]['edge_mlp'][0][0]
    Wa0, Wb0 = W1_0[:D], W1_0[D:2 * D]
    row_spec = pl.BlockSpec((TN, D), lambda i: (i, 0))
    wspec = pl.BlockSpec((D, D), lambda i: (0, 0))
    bspec = pl.BlockSpec((1, D), lambda i: (0, 0))
    h, hA, hB = pl.pallas_call(
        _enc_body,
        grid=(n_grid,),
        in_specs=[row_spec] + [wspec, bspec] * 3 + [wspec, wspec],
        out_specs=(row_spec, row_spec, row_spec),
        out_shape=(jax.ShapeDtypeStruct((n_nodes, D), jnp.float32),) * 3,
    )(x, A1, b2(a1), A2, b2(a2), A3, b2(a3), Wa0, Wb0)

    (E1, e1), (E2, e2), (E3, e3) = params['edge_enc']
    k_in = E1.shape[0]
    k_pad = 8
    ea8 = jnp.pad(edge_attr, ((0, 0), (0, k_pad - k_in)))
    E1p = jnp.pad(E1, ((0, k_pad - k_in), (0, 0)))
    erow_spec = pl.BlockSpec((TE, D), lambda i: (i, 0))
    ein_spec = pl.BlockSpec((TE, k_pad), lambda i: (i, 0))
    ew_spec = pl.BlockSpec((k_pad, D), lambda i: (0, 0))
    e = pl.pallas_call(
        _edge_enc_body,
        grid=(e_grid,),
        in_specs=[ein_spec, ew_spec, bspec, wspec, bspec, wspec, bspec],
        out_specs=erow_spec,
        out_shape=jax.ShapeDtypeStruct((E, D), jnp.bfloat16),
    )(ea8, E1p, b2(e1), E2, b2(e2), E3, b2(e3))

    n_pad = ((n_nodes + 8 * NS - 1) // (8 * NS)) * (8 * NS)
    ones_kw = jnp.ones((KCH, WC), jnp.float32)
    zeros_nw = jnp.zeros((n_pad, WC), jnp.float32)
    zeros_nd = jnp.zeros((n_pad, D), jnp.float32)
    cnt = _sc_counts(dst, ones_kw, zeros_nw)

    spart_spec = pl.BlockSpec((NC, TN, D), lambda i: (0, i, 0))
    cnt_spec = pl.BlockSpec((NC, TN, WC), lambda i: (0, i, 0))
    perm = jnp.array(
        [32 * (l // 16) + l % 16 for l in range(D // 2)]
        + [32 * (l // 16) + 16 + l % 16 for l in range(D // 2)], jnp.int32)
    g_spec = pl.BlockSpec((TE, D // 2), lambda i: (i, 0))

    for bi, blk in enumerate(blocks):
        (W1, w_b1), (W2, w_b2), (W3, w_b3) = blk['edge_mlp']
        W1c = W1[2 * D:]
        g3 = _sc_gather(hA, hB, src, dst)
        msg = pl.pallas_call(
            _edge_mlp_body,
            grid=(e_grid,),
            in_specs=[g_spec, erow_spec,
                      wspec, bspec, wspec, bspec, wspec, bspec],
            out_specs=erow_spec,
            out_shape=jax.ShapeDtypeStruct((E, D), jnp.float32),
        )(g3, e, W1c[:, perm], b2(w_b1)[:, perm], W2[perm, :],
          b2(w_b2), W3, b2(w_b3))

        S = _sc_scatter(msg, dst, zeros_nd)

        (V1, c1), (V2, c2), (V3, c3) = blk['node_mlp']
        V1a, V1b = V1[:D], V1[D:]
        if bi + 1 < len(blocks):
            W1n = blocks[bi + 1]['edge_mlp'][0][0]
            Wan, Wbn = W1n[:D], W1n[D:2 * D]
            h, hA, hB = pl.pallas_call(
                _node_mid_body,
                grid=(n_grid,),
                in_specs=[row_spec, spart_spec, cnt_spec,
                          wspec, wspec, bspec, wspec, bspec, wspec, bspec,
                          wspec, wspec],
                out_specs=(row_spec, row_spec, row_spec),
                out_shape=(jax.ShapeDtypeStruct((n_nodes, D), jnp.float32),) * 3,
            )(h, S, cnt, V1a, V1b, b2(c1), V2, b2(c2), V3, b2(c3), Wan, Wbn)
        else:
            (U1, d1), (U2, d2), (U3, d3) = params['decoder']
            out_dim = U3.shape[1]
            U3p = jnp.pad(U3, ((0, 0), (0, D - out_dim)))
            d3p = jnp.pad(d3, ((0, D - out_dim),))
            dec = pl.pallas_call(
                _node_last_body,
                grid=(n_grid,),
                in_specs=[row_spec, spart_spec, cnt_spec,
                          wspec, wspec, bspec, wspec, bspec, wspec, bspec,
                          wspec, bspec, wspec, bspec, wspec, bspec],
                out_specs=row_spec,
                out_shape=jax.ShapeDtypeStruct((n_nodes, D), jnp.float32),
            )(h, S, cnt, V1a, V1b, b2(c1), V2, b2(c2), V3, b2(c3),
              U1, b2(d1), U2, b2(d2), U3p, b2(d3p))

    return dec[:, :out_dim]
